# baseline (speedup 1.0000x reference)
import functools

import jax
import jax.numpy as jnp
from jax.experimental import pallas as pl
from jax.experimental.pallas import tpu as pltpu

_EPS = 1e-5
_INV_SQRT2 = 0.7071067811865476


def _body(x_ref, src_ref, dst_ref, gamma_ref, beta_ref, wcat_ref, bv_ref,
          b1_ref, w2_ref, b2_ref, out_ref, v_scr, a_scr, c_scr, acc_scr,
          *, n_pad, n_edges, d, gh, bt):
    gamma = gamma_ref[...]
    beta = beta_ref[...]
    wcat = wcat_ref[...]
    bv = bv_ref[...]
    b1 = b1_ref[...]
    w2 = w2_ref[...]
    b2 = b2_ref[0]

    x = x_ref[...]
    mu = jnp.mean(x, axis=2, keepdims=True)
    xc = x - mu
    var = jnp.mean(xc * xc, axis=2, keepdims=True)
    xhat = (xc * jax.lax.rsqrt(var + _EPS) * gamma.reshape(1, 1, d)
            + beta.reshape(1, 1, d))
    xt = jnp.transpose(xhat, (1, 0, 2))
    y = jnp.dot(xt.reshape(n_pad * bt, d), wcat,
                preferred_element_type=jnp.float32)
    y3 = y.reshape(n_pad, bt, d + 2 * gh)
    v_scr[...] = y3[:, :, :d] + bv.reshape(1, 1, d)
    a_scr[...] = y3[:, :, d:d + gh]
    c_scr[...] = y3[:, :, d + gh:] + b1.reshape(1, 1, gh)

    acc_scr[...] = jnp.zeros_like(acc_scr)

    n_banks = acc_scr.shape[0] // n_pad
    w2b = w2.reshape(1, 1, gh)
    for e in range(n_edges):
        s = src_ref[e]
        t = dst_ref[e]
        tb = t + (e % n_banks) * n_pad
        hin = a_scr[pl.ds(s, 1)] + c_scr[pl.ds(t, 1)]
        gel = 0.5 * hin * (1.0 + jax.lax.erf(hin * _INV_SQRT2))
        u = jnp.sum(gel * w2b, axis=2, keepdims=True) + b2
        gate = jax.nn.sigmoid(u)
        acc_scr[pl.ds(tb, 1)] = (acc_scr[pl.ds(tb, 1)]
                                 + gate * v_scr[pl.ds(s, 1)])

    acc = acc_scr[...].reshape(n_banks, n_pad, bt, d).sum(axis=0)
    out_ref[...] = x + jnp.transpose(acc, (1, 0, 2))


def kernel(h, src_index, dst_index, gamma, beta, Wv, bv, W1, b1, W2, b2):
    B, N, d = h.shape
    E = src_index.shape[0]
    gh = W1.shape[0]
    n_pad = (N + 7) // 8 * 8

    wcat = jnp.concatenate([Wv.T, W1[:, :d].T, W1[:, d:].T], axis=1)

    bt = 256
    grid = (B // bt,)

    body = functools.partial(_body, n_pad=n_pad, n_edges=E, d=d, gh=gh, bt=bt)
    out = pl.pallas_call(
        body,
        grid=grid,
        in_specs=[
            pl.BlockSpec((bt, n_pad, d), lambda i: (i, 0, 0)),
            pl.BlockSpec(memory_space=pltpu.SMEM),
            pl.BlockSpec(memory_space=pltpu.SMEM),
            pl.BlockSpec(memory_space=pltpu.VMEM),
            pl.BlockSpec(memory_space=pltpu.VMEM),
            pl.BlockSpec(memory_space=pltpu.VMEM),
            pl.BlockSpec(memory_space=pltpu.VMEM),
            pl.BlockSpec(memory_space=pltpu.VMEM),
            pl.BlockSpec(memory_space=pltpu.VMEM),
            pl.BlockSpec(memory_space=pltpu.SMEM),
        ],
        out_specs=pl.BlockSpec((bt, n_pad, d), lambda i: (i, 0, 0)),
        out_shape=jax.ShapeDtypeStruct((B, N, d), jnp.float32),
        scratch_shapes=[
            pltpu.VMEM((n_pad, bt, d), jnp.float32),
            pltpu.VMEM((n_pad, bt, gh), jnp.float32),
            pltpu.VMEM((n_pad, bt, gh), jnp.float32),
            pltpu.VMEM((4 * n_pad, bt, d), jnp.float32),
        ],
    )(
        h,
        src_index,
        dst_index,
        gamma.reshape(1, d),
        beta.reshape(1, d),
        wcat,
        bv.reshape(1, d),
        b1.reshape(1, gh),
        W2.reshape(1, gh),
        b2,
    )
    return out


# speedup vs baseline: 1.1312x; 1.1312x over previous
import functools

import jax
import jax.numpy as jnp
from jax.experimental import pallas as pl
from jax.experimental.pallas import tpu as pltpu

_EPS = 1e-5
_INV_SQRT2 = 0.7071067811865476
_SQRT2 = 1.4142135623730951


def _body(x_ref, src_ref, dst_ref, gamma_ref, beta_ref, wcat_ref, bv_ref,
          b1_ref, w2_ref, b2_ref, out_ref, v_scr, a_scr, c_scr, acc_scr,
          *, n_pad, n_edges, d, gh, bt):
    gamma = gamma_ref[...]
    beta = beta_ref[...]
    wcat = wcat_ref[...]
    bv = bv_ref[...]
    b1 = b1_ref[...]
    w2 = w2_ref[...]
    b2 = b2_ref[0]

    x = x_ref[...]
    mu = jnp.mean(x, axis=2, keepdims=True)
    xc = x - mu
    var = jnp.mean(xc * xc, axis=2, keepdims=True)
    xhat = (xc * jax.lax.rsqrt(var + _EPS) * gamma.reshape(1, 1, d)
            + beta.reshape(1, 1, d))
    xt = jnp.transpose(xhat, (1, 0, 2))
    y = jnp.dot(xt.reshape(n_pad * bt, d), wcat,
                preferred_element_type=jnp.float32)
    y3 = y.reshape(n_pad, bt, d + 2 * gh)
    v_scr[...] = y3[:, :, :d] + bv.reshape(1, 1, d)
    a_scr[...] = y3[:, :, d:d + gh]
    c_scr[...] = y3[:, :, d + gh:] + b1.reshape(1, 1, gh)

    acc_scr[...] = jnp.zeros_like(acc_scr)

    w2b = w2.reshape(1, 1, gh)
    for e in range(n_edges):
        s = src_ref[e]
        t = dst_ref[e]
        hin = a_scr[pl.ds(s, 1)] + c_scr[pl.ds(t, 1)]
        gel = hin * (1.0 + jax.lax.erf(hin))
        u = jnp.sum(gel * w2b, axis=2, keepdims=True) + b2
        gate = jax.nn.sigmoid(u)
        acc_scr[pl.ds(t, 1)] = (acc_scr[pl.ds(t, 1)]
                                + gate * v_scr[pl.ds(s, 1)])

    out_ref[...] = x + jnp.transpose(acc_scr[...], (1, 0, 2))


def kernel(h, src_index, dst_index, gamma, beta, Wv, bv, W1, b1, W2, b2):
    B, N, d = h.shape
    E = src_index.shape[0]
    gh = W1.shape[0]
    n_pad = (N + 7) // 8 * 8

    wcat = jnp.concatenate(
        [Wv.T, W1[:, :d].T * _INV_SQRT2, W1[:, d:].T * _INV_SQRT2], axis=1)
    w2s = W2.reshape(1, gh) * (_SQRT2 * 0.5)

    bt = 512
    grid = (B // bt,)

    body = functools.partial(_body, n_pad=n_pad, n_edges=E, d=d, gh=gh, bt=bt)
    out = pl.pallas_call(
        body,
        grid=grid,
        in_specs=[
            pl.BlockSpec((bt, n_pad, d), lambda i: (i, 0, 0)),
            pl.BlockSpec(memory_space=pltpu.SMEM),
            pl.BlockSpec(memory_space=pltpu.SMEM),
            pl.BlockSpec(memory_space=pltpu.VMEM),
            pl.BlockSpec(memory_space=pltpu.VMEM),
            pl.BlockSpec(memory_space=pltpu.VMEM),
            pl.BlockSpec(memory_space=pltpu.VMEM),
            pl.BlockSpec(memory_space=pltpu.VMEM),
            pl.BlockSpec(memory_space=pltpu.VMEM),
            pl.BlockSpec(memory_space=pltpu.SMEM),
        ],
        out_specs=pl.BlockSpec((bt, n_pad, d), lambda i: (i, 0, 0)),
        out_shape=jax.ShapeDtypeStruct((B, N, d), jnp.float32),
        scratch_shapes=[
            pltpu.VMEM((n_pad, bt, d), jnp.float32),
            pltpu.VMEM((n_pad, bt, gh), jnp.float32),
            pltpu.VMEM((n_pad, bt, gh), jnp.float32),
            pltpu.VMEM((n_pad, bt, d), jnp.float32),
        ],
    )(
        h,
        src_index,
        dst_index,
        gamma.reshape(1, d),
        beta.reshape(1, d),
        wcat,
        bv.reshape(1, d),
        b1.reshape(1, gh) * _INV_SQRT2,
        w2s,
        b2,
    )
    return out


# speedup vs baseline: 1.1531x; 1.0193x over previous
import functools

import jax
import jax.numpy as jnp
from jax.experimental import pallas as pl
from jax.experimental.pallas import tpu as pltpu

_EPS = 1e-5
_INV_SQRT2 = 0.7071067811865476
_SQRT2 = 1.4142135623730951


def _body(x_ref, src_ref, dst_ref, gamma_ref, beta_ref, wcat_ref, bv_ref,
          b1_ref, w2_ref, b2_ref, out_ref, v_scr, a_scr, c_scr, acc_scr,
          *, n_nodes, n_edges, d, gh, bt):
    gamma = gamma_ref[...]
    beta = beta_ref[...]
    wcat = wcat_ref[...]
    bv = bv_ref[...]
    b1 = b1_ref[...]
    w2 = w2_ref[...]
    b2 = b2_ref[0]

    x = x_ref[...]
    mu = jnp.mean(x, axis=2, keepdims=True)
    xc = x - mu
    var = jnp.mean(xc * xc, axis=2, keepdims=True)
    xhat = (xc * jax.lax.rsqrt(var + _EPS) * gamma.reshape(1, 1, d)
            + beta.reshape(1, 1, d))
    xt = jnp.transpose(xhat, (1, 0, 2))
    y = jnp.dot(xt.reshape(n_nodes * bt, d), wcat,
                preferred_element_type=jnp.float32)
    y3 = y.reshape(n_nodes, bt, d + 2 * gh)
    v_scr[...] = y3[:, :, :d] + bv.reshape(1, 1, d)
    a_scr[...] = y3[:, :, d:d + gh]
    c_scr[...] = y3[:, :, d + gh:] + b1.reshape(1, 1, gh)

    acc_scr[...] = jnp.zeros_like(acc_scr)

    w2b = w2.reshape(1, 1, gh)
    for e in range(n_edges):
        s = src_ref[e]
        t = dst_ref[e]
        hin = a_scr[pl.ds(s, 1)] + c_scr[pl.ds(t, 1)]
        gel = hin * (1.0 + jax.lax.erf(hin))
        u = jnp.sum(gel * w2b, axis=2, keepdims=True) + b2
        gate = jax.nn.sigmoid(u)
        acc_scr[pl.ds(t, 1)] = (acc_scr[pl.ds(t, 1)]
                                + gate * v_scr[pl.ds(s, 1)])

    out_ref[...] = x + jnp.transpose(acc_scr[...], (1, 0, 2))


def kernel(h, src_index, dst_index, gamma, beta, Wv, bv, W1, b1, W2, b2):
    B, N, d = h.shape
    E = src_index.shape[0]
    gh = W1.shape[0]

    wcat = jnp.concatenate(
        [Wv.T, W1[:, :d].T * _INV_SQRT2, W1[:, d:].T * _INV_SQRT2], axis=1)
    w2s = W2.reshape(1, gh) * (_SQRT2 * 0.5)

    bt = 512
    grid = (B // bt,)

    body = functools.partial(_body, n_nodes=N, n_edges=E, d=d, gh=gh, bt=bt)
    out = pl.pallas_call(
        body,
        grid=grid,
        in_specs=[
            pl.BlockSpec((bt, N, d), lambda i: (i, 0, 0)),
            pl.BlockSpec(memory_space=pltpu.SMEM),
            pl.BlockSpec(memory_space=pltpu.SMEM),
            pl.BlockSpec(memory_space=pltpu.VMEM),
            pl.BlockSpec(memory_space=pltpu.VMEM),
            pl.BlockSpec(memory_space=pltpu.VMEM),
            pl.BlockSpec(memory_space=pltpu.VMEM),
            pl.BlockSpec(memory_space=pltpu.VMEM),
            pl.BlockSpec(memory_space=pltpu.VMEM),
            pl.BlockSpec(memory_space=pltpu.SMEM),
        ],
        out_specs=pl.BlockSpec((bt, N, d), lambda i: (i, 0, 0)),
        out_shape=jax.ShapeDtypeStruct((B, N, d), jnp.float32),
        scratch_shapes=[
            pltpu.VMEM((N, bt, d), jnp.float32),
            pltpu.VMEM((N, bt, gh), jnp.float32),
            pltpu.VMEM((N, bt, gh), jnp.float32),
            pltpu.VMEM((N, bt, d), jnp.float32),
        ],
    )(
        h,
        src_index,
        dst_index,
        gamma.reshape(1, d),
        beta.reshape(1, d),
        wcat,
        bv.reshape(1, d),
        b1.reshape(1, gh) * _INV_SQRT2,
        w2s,
        b2,
    )
    return out


# speedup vs baseline: 2.5703x; 2.2291x over previous
import functools

import jax
import jax.numpy as jnp
from jax.experimental import pallas as pl
from jax.experimental.pallas import tpu as pltpu

_EPS = 1e-5
_INV_SQRT2 = 0.7071067811865476
_SQRT2 = 1.4142135623730951


def _body(x_ref, src_ref, dst_ref, gamma_ref, beta_ref, wcat_ref, bv_ref,
          b1_ref, w2_ref, b2_ref, out_ref, v_scr, a_scr, c_scr,
          *, n_nodes, n_edges, d, gh, bt):
    gamma = gamma_ref[...]
    beta = beta_ref[...]
    wcat = wcat_ref[...]
    bv = bv_ref[...]
    b1 = b1_ref[...]
    w2 = w2_ref[...]
    b2 = b2_ref[0]

    x = x_ref[...]
    mu = jnp.mean(x, axis=2, keepdims=True)
    xc = x - mu
    var = jnp.mean(xc * xc, axis=2, keepdims=True)
    xhat = (xc * jax.lax.rsqrt(var + _EPS) * gamma.reshape(1, 1, d)
            + beta.reshape(1, 1, d))
    y = jnp.dot(xhat.reshape(n_nodes * bt, d), wcat,
                preferred_element_type=jnp.float32)
    y3 = y.reshape(n_nodes, bt, d + 2 * gh)
    v_scr[...] = y3[:, :, :d] + bv.reshape(1, 1, d)
    a_scr[...] = y3[:, :, d:d + gh]
    c_scr[...] = y3[:, :, d + gh:] + b1.reshape(1, 1, gh)

    out_ref[...] = x

    w2b = w2.reshape(1, 1, gh)
    for e in range(n_edges):
        s = src_ref[e]
        t = dst_ref[e]
        hin = a_scr[pl.ds(s, 1)] + c_scr[pl.ds(t, 1)]
        gel = hin * (1.0 + jax.lax.erf(hin))
        u = jnp.sum(gel * w2b, axis=2, keepdims=True) + b2
        gate = jax.nn.sigmoid(u)
        out_ref[pl.ds(t, 1)] = (out_ref[pl.ds(t, 1)]
                                + gate * v_scr[pl.ds(s, 1)])


def kernel(h, src_index, dst_index, gamma, beta, Wv, bv, W1, b1, W2, b2):
    B, N, d = h.shape
    E = src_index.shape[0]
    gh = W1.shape[0]

    wcat = jnp.concatenate(
        [Wv.T, W1[:, :d].T * _INV_SQRT2, W1[:, d:].T * _INV_SQRT2], axis=1)
    w2s = W2.reshape(1, gh) * (_SQRT2 * 0.5)

    ht = jnp.transpose(h, (1, 0, 2))

    bt = 512
    grid = (B // bt,)

    body = functools.partial(_body, n_nodes=N, n_edges=E, d=d, gh=gh, bt=bt)
    out = pl.pallas_call(
        body,
        grid=grid,
        in_specs=[
            pl.BlockSpec((N, bt, d), lambda i: (0, i, 0)),
            pl.BlockSpec(memory_space=pltpu.SMEM),
            pl.BlockSpec(memory_space=pltpu.SMEM),
            pl.BlockSpec(memory_space=pltpu.VMEM),
            pl.BlockSpec(memory_space=pltpu.VMEM),
            pl.BlockSpec(memory_space=pltpu.VMEM),
            pl.BlockSpec(memory_space=pltpu.VMEM),
            pl.BlockSpec(memory_space=pltpu.VMEM),
            pl.BlockSpec(memory_space=pltpu.VMEM),
            pl.BlockSpec(memory_space=pltpu.SMEM),
        ],
        out_specs=pl.BlockSpec((N, bt, d), lambda i: (0, i, 0)),
        out_shape=jax.ShapeDtypeStruct((N, B, d), jnp.float32),
        scratch_shapes=[
            pltpu.VMEM((N, bt, d), jnp.float32),
            pltpu.VMEM((N, bt, gh), jnp.float32),
            pltpu.VMEM((N, bt, gh), jnp.float32),
        ],
    )(
        ht,
        src_index,
        dst_index,
        gamma.reshape(1, d),
        beta.reshape(1, d),
        wcat,
        bv.reshape(1, d),
        b1.reshape(1, gh) * _INV_SQRT2,
        w2s,
        b2,
    )
    return jnp.transpose(out, (1, 0, 2))
